# baseline (speedup 1.0000x reference)
import jax
import jax.numpy as jnp
from jax import lax
from jax.experimental import pallas as pl
from jax.experimental.pallas import tpu as pltpu
from jax.experimental.pallas import tpu_sc as plsc

NC = 2
NS = 16
L = 16
NW = NC * NS

B = 16384
N = 128
BPW = B // NW
CH = 128
NCH = BPW // CH
NBUF = 2
UNR = 4


def _sc_body(x_hbm, ids_hbm, c_hbm, out_hbm, idx_v, rows_v, x_v, out_v,
             sems_g, sems_x):
    wid = lax.axis_index("s") * NC + lax.axis_index("c")
    base = wid * BPW
    pltpu.async_copy(x_hbm.at[pl.ds(base, CH)], x_v.at[0], sems_x.at[0])
    pltpu.sync_copy(ids_hbm.at[pl.ds(base, BPW)], idx_v)
    lane = lax.iota(jnp.int32, L)

    def start(c, first=False):
        s = c % NBUF
        pltpu.async_copy(c_hbm.at[idx_v.at[pl.ds(c * CH, CH)]],
                         rows_v.at[s], sems_g.at[s])
        if not first:
            pltpu.async_copy(x_hbm.at[pl.ds(base + c * CH, CH)],
                             x_v.at[s], sems_x.at[s])

    def wait(c):
        s = c % NBUF
        pltpu.make_async_copy(c_hbm.at[idx_v.at[pl.ds(c * CH, CH)]],
                              rows_v.at[s], sems_g.at[s]).wait()
        pltpu.make_async_copy(x_hbm.at[pl.ds(base + c * CH, CH)],
                              x_v.at[s], sems_x.at[s]).wait()

    start(0, first=True)
    for c in range(NCH):
        if c + 1 < NCH:
            start(c + 1)
        wait(c)
        s = c % NBUF

        def grp_body(g, _):
            def pair_body(t, acc):
                for u in range(UNR):
                    j = t * UNR + u
                    r = g * L + j
                    v = rows_v[s, r, pl.ds(0, L)] * x_v[s, r, pl.ds(0, L)]
                    for k in range(1, N // L):
                        v = v + (rows_v[s, r, pl.ds(k * L, L)]
                                 * x_v[s, r, pl.ds(k * L, L)])
                    for sh in (1, 2, 4, 8):
                        v = v + v.at[lane ^ sh].get(
                            mode="promise_in_bounds")
                    acc = jnp.where(lane == j, v, acc)
                return acc

            acc = lax.fori_loop(0, L // UNR, pair_body,
                                jnp.zeros((L,), jnp.float32))
            off = pl.multiple_of(c * CH + g * L, L)
            out_v[pl.ds(off, L)] = acc
            return 0

        lax.fori_loop(0, CH // L, grp_body, 0)

    pltpu.sync_copy(out_v, out_hbm.at[pl.ds(base, BPW)])


def kernel(x, task_ids, C):
    mesh = plsc.VectorSubcoreMesh(core_axis_name="c", subcore_axis_name="s",
                                  num_cores=NC, num_subcores=NS)
    return pl.kernel(
        _sc_body,
        out_type=jax.ShapeDtypeStruct((B,), jnp.float32),
        mesh=mesh,
        compiler_params=pltpu.CompilerParams(needs_layout_passes=False),
        scratch_types=[
            pltpu.VMEM((BPW,), jnp.int32),
            pltpu.VMEM((NBUF, CH, N), jnp.float32),
            pltpu.VMEM((NBUF, CH, N), jnp.float32),
            pltpu.VMEM((BPW,), jnp.float32),
            pltpu.SemaphoreType.DMA((NBUF,)),
            pltpu.SemaphoreType.DMA((NBUF,)),
        ],
    )(x, task_ids.astype(jnp.int32), C)


# speedup vs baseline: 1.0234x; 1.0234x over previous
import jax
import jax.numpy as jnp
from jax import lax
from jax.experimental import pallas as pl
from jax.experimental.pallas import tpu as pltpu
from jax.experimental.pallas import tpu_sc as plsc

NC = 2
NS = 16
L = 16
NW = NC * NS

B = 16384
N = 128
BPW = B // NW
CH = 128
NCH = BPW // CH
NBUF = 2
UNR = 2


def _sc_body(x_hbm, ids_hbm, c_hbm, out_hbm, idx_v, rows_v, x_v, out_v,
             sems_g, sems_x):
    wid = lax.axis_index("s") * NC + lax.axis_index("c")
    base = wid * BPW
    pltpu.async_copy(x_hbm.at[pl.ds(base, CH)], x_v.at[0], sems_x.at[0])
    pltpu.sync_copy(ids_hbm.at[pl.ds(base, BPW)], idx_v)
    lane = lax.iota(jnp.int32, L)

    def start(c, first=False):
        s = c % NBUF
        pltpu.async_copy(c_hbm.at[idx_v.at[pl.ds(c * CH, CH)]],
                         rows_v.at[s], sems_g.at[s])
        if not first:
            pltpu.async_copy(x_hbm.at[pl.ds(base + c * CH, CH)],
                             x_v.at[s], sems_x.at[s])

    def wait(c):
        s = c % NBUF
        pltpu.make_async_copy(c_hbm.at[idx_v.at[pl.ds(c * CH, CH)]],
                              rows_v.at[s], sems_g.at[s]).wait()
        pltpu.make_async_copy(x_hbm.at[pl.ds(base + c * CH, CH)],
                              x_v.at[s], sems_x.at[s]).wait()

    start(0, first=True)
    for c in range(NCH):
        if c + 1 < NCH:
            start(c + 1)
        wait(c)
        s = c % NBUF

        @plsc.parallel_loop(0, CH // L)
        def grp_body(g):
            def pair_body(t, acc):
                for u in range(UNR):
                    j = t * UNR + u
                    r = g * L + j
                    parts = [rows_v[s, r, pl.ds(k * L, L)]
                             * x_v[s, r, pl.ds(k * L, L)]
                             for k in range(N // L)]
                    while len(parts) > 1:
                        parts = [parts[i] + parts[i + 1]
                                 for i in range(0, len(parts), 2)]
                    v = parts[0]
                    for sh in (1, 2, 4, 8):
                        v = v + v.at[lane ^ sh].get(
                            mode="promise_in_bounds")
                    acc = jnp.where(lane == j, v, acc)
                return acc

            acc = lax.fori_loop(0, L // UNR, pair_body,
                                jnp.zeros((L,), jnp.float32))
            off = pl.multiple_of(c * CH + g * L, L)
            out_v[pl.ds(off, L)] = acc

    pltpu.sync_copy(out_v, out_hbm.at[pl.ds(base, BPW)])


def kernel(x, task_ids, C):
    mesh = plsc.VectorSubcoreMesh(core_axis_name="c", subcore_axis_name="s",
                                  num_cores=NC, num_subcores=NS)
    return pl.kernel(
        _sc_body,
        out_type=jax.ShapeDtypeStruct((B,), jnp.float32),
        mesh=mesh,
        compiler_params=pltpu.CompilerParams(needs_layout_passes=False),
        scratch_types=[
            pltpu.VMEM((BPW,), jnp.int32),
            pltpu.VMEM((NBUF, CH, N), jnp.float32),
            pltpu.VMEM((NBUF, CH, N), jnp.float32),
            pltpu.VMEM((BPW,), jnp.float32),
            pltpu.SemaphoreType.DMA((NBUF,)),
            pltpu.SemaphoreType.DMA((NBUF,)),
        ],
    )(x, task_ids.astype(jnp.int32), C)


# speedup vs baseline: 1.0294x; 1.0058x over previous
import jax
import jax.numpy as jnp
from jax import lax
from jax.experimental import pallas as pl
from jax.experimental.pallas import tpu as pltpu
from jax.experimental.pallas import tpu_sc as plsc

NC = 2
NS = 16
L = 16
NW = NC * NS

B = 16384
N = 128
BPW = B // NW
CH = 128
NCH = BPW // CH
NBUF = 2
UNR = 2


def _sc_body(x_hbm, ids_hbm, c_hbm, out_hbm, idx_v, rows_v, x_v, out_v,
             sems_g, sems_x):
    wid = lax.axis_index("s") * NC + lax.axis_index("c")
    base = wid * BPW
    pltpu.async_copy(x_hbm.at[pl.ds(base, CH)], x_v.at[0], sems_x.at[0])
    pltpu.sync_copy(ids_hbm.at[pl.ds(base, BPW)], idx_v)
    lane = lax.iota(jnp.int32, L)

    def start(c, first=False):
        s = c % NBUF
        pltpu.async_copy(c_hbm.at[idx_v.at[pl.ds(c * CH, CH)]],
                         rows_v.at[s], sems_g.at[s])
        if not first:
            pltpu.async_copy(x_hbm.at[pl.ds(base + c * CH, CH)],
                             x_v.at[s], sems_x.at[s])

    def wait(c):
        s = c % NBUF
        pltpu.make_async_copy(c_hbm.at[idx_v.at[pl.ds(c * CH, CH)]],
                              rows_v.at[s], sems_g.at[s]).wait()
        pltpu.make_async_copy(x_hbm.at[pl.ds(base + c * CH, CH)],
                              x_v.at[s], sems_x.at[s]).wait()

    start(0, first=True)
    for c in range(NCH):
        if c + 1 < NCH:
            start(c + 1)
        wait(c)
        s = c % NBUF

        @plsc.parallel_loop(0, CH, unroll=UNR)
        def row_body(r):
            parts = [rows_v[s, r, pl.ds(k * L, L)]
                     * x_v[s, r, pl.ds(k * L, L)]
                     for k in range(N // L)]
            while len(parts) > 1:
                parts = [parts[i] + parts[i + 1]
                         for i in range(0, len(parts), 2)]
            v = parts[0]
            for sh in (1, 2, 4, 8):
                v = v + v.at[lane ^ sh].get(mode="promise_in_bounds")
            plsc.store_scatter(out_v,
                               [jnp.full((L,), c * CH, jnp.int32) + r],
                               v, mask=lane == 0)

    pltpu.sync_copy(out_v, out_hbm.at[pl.ds(base, BPW)])


def kernel(x, task_ids, C):
    mesh = plsc.VectorSubcoreMesh(core_axis_name="c", subcore_axis_name="s",
                                  num_cores=NC, num_subcores=NS)
    return pl.kernel(
        _sc_body,
        out_type=jax.ShapeDtypeStruct((B,), jnp.float32),
        mesh=mesh,
        compiler_params=pltpu.CompilerParams(needs_layout_passes=False),
        scratch_types=[
            pltpu.VMEM((BPW,), jnp.int32),
            pltpu.VMEM((NBUF, CH, N), jnp.float32),
            pltpu.VMEM((NBUF, CH, N), jnp.float32),
            pltpu.VMEM((BPW,), jnp.float32),
            pltpu.SemaphoreType.DMA((NBUF,)),
            pltpu.SemaphoreType.DMA((NBUF,)),
        ],
    )(x, task_ids.astype(jnp.int32), C)
